# baseline (speedup 1.0000x reference)
import functools

import jax
import jax.numpy as jnp
from jax import lax
from jax.experimental import pallas as pl
from jax.experimental.pallas import tpu as pltpu
from jax.experimental.pallas import tpu_sc as plsc

N_NODES = 10000
N_EDGES = 320000
D_FEAT = 128
D_EDGE = 16
N_GRAPHS = 16
D_GLOBAL = 32
H1 = 256
H2 = 256
N_OUT = 128

BN = 2000

NC = 2
NS = 16
NW = NC * NS
EPR = 128
N_ROWS = N_EDGES // EPR
K_FULL = N_ROWS // NW
N_EXTRA = N_ROWS - K_FULL * NW
NPT = 1000


def _sc_scatter_build():
    mesh = plsc.VectorSubcoreMesh(core_axis_name="c", subcore_axis_name="s")
    NBUF = 6

    @functools.partial(
        pl.kernel, mesh=mesh,
        compiler_params=pltpu.CompilerParams(
            use_tc_tiling_on_sc=False, needs_layout_passes=False),
        out_type=(jax.ShapeDtypeStruct((N_NODES, 128), jnp.float32),
                  jax.ShapeDtypeStruct((N_NODES, 128), jnp.float32)),
        scratch_types=(
            [pltpu.VMEM((EPR,), jnp.int32) for _ in range(NBUF)]
            + [pltpu.VMEM((2, 8, EPR), jnp.float32) for _ in range(NBUF)]
            + [pltpu.VMEM((EPR, D_EDGE), jnp.float32) for _ in range(NBUF)]
            + [pltpu.VMEM((NPT, D_EDGE), jnp.float32),
               pltpu.VMEM_SHARED((N_NODES, D_EDGE), jnp.float32)]
            + [pltpu.SemaphoreType.DMA for _ in range(2 * NBUF)]
        ))
    def sc_scatter(ev_hbm, ei_hbm, out0_hbm, out1_hbm, *bufs):
        idx = bufs[0:NBUF]
        tb = bufs[NBUF:2 * NBUF]
        sb = bufs[2 * NBUF:3 * NBUF]
        stage = bufs[3 * NBUF]
        acc = bufs[3 * NBUF + 1]
        sl = bufs[3 * NBUF + 2:3 * NBUF + 2 + NBUF]
        ss = bufs[3 * NBUF + 2 + NBUF:3 * NBUF + 2 + 2 * NBUF]

        c = lax.axis_index("c")
        s = lax.axis_index("s")
        w = s * NC + c
        iota16 = lax.broadcasted_iota(jnp.int32, (D_EDGE,), 0)

        def zrow(i, carry):
            stage[i, :] = jnp.zeros((D_EDGE,), jnp.float32)
            return carry
        lax.fori_loop(0, NPT, zrow, 0)

        @pl.when(s < N_NODES // NPT)
        def _():
            pltpu.sync_copy(stage, acc.at[pl.ds(s * NPT, NPT)])
        plsc.subcore_barrier()

        def start_load(k, b):
            r = w + k * NW
            pltpu.async_copy(ei_hbm.at[r, 1], idx[b], sl[b])
            pltpu.async_copy(ev_hbm.at[:, r], tb[b], sl[b])

        def wait_load(b):
            pltpu.make_async_copy(ei_hbm.at[0, 1], idx[b], sl[b]).wait()
            pltpu.make_async_copy(ev_hbm.at[:, 0], tb[b], sl[b]).wait()

        rot = [((iota16 + d) & 15) for d in range(D_EDGE)]
        hi3 = iota16 >> 3
        lo3 = iota16 & 7
        zv = jnp.zeros((D_EDGE,), jnp.int32)
        ldb = [hi3 * (8 * EPR) + lo3 * EPR + rot[d] for d in range(D_EDGE)]
        stb = [rot[d] * D_EDGE + iota16 for d in range(D_EDGE)]

        def transpose(b):
            def tr(m, carry):
                vs = [plsc.load_gather(tb[b], [zv, zv, ldb[d] + m * D_EDGE])
                      for d in range(D_EDGE)]
                for d in range(D_EDGE):
                    plsc.store_scatter(
                        sb[b], [zv, stb[d] + m * (D_EDGE * D_EDGE)], vs[d])
                return carry
            lax.fori_loop(0, EPR // D_EDGE, tr, 0)

        def start_scat(b):
            pltpu.async_copy(sb[b], acc.at[idx[b]], ss[b], add=True)

        def wait_scat(b):
            pltpu.make_async_copy(sb[b], acc.at[idx[b]], ss[b]).wait()

        for b in range(3):
            start_load(b, b)

        def body(g, carry):
            for i in range(NBUF):
                b = i
                k = g * NBUF + i

                wait_load(b)
                transpose(b)
                start_scat(b)

                bb = (i + 3) % NBUF

                @pl.when(k + 3 < K_FULL)
                def _():
                    @pl.when(k >= 3)
                    def _():
                        wait_scat(bb)
                    start_load(k + 3, bb)
            return carry

        lax.fori_loop(0, K_FULL // NBUF, body, 0)

        @pl.when(w < N_EXTRA)
        def _():
            wait_scat(0)
            start_load(K_FULL, 0)
            wait_load(0)
            transpose(0)
            start_scat(0)

        for b in range(NBUF):
            wait_scat(b)

        plsc.subcore_barrier()

        @pl.when(s < N_NODES // NPT)
        def _():
            src_slice = acc.at[pl.ds(s * NPT, NPT)]

            @pl.when(c == 0)
            def _():
                pltpu.sync_copy(src_slice,
                                out0_hbm.at[pl.ds(s * NPT, NPT), pl.ds(0, D_EDGE)])

            @pl.when(c == 1)
            def _():
                pltpu.sync_copy(src_slice,
                                out1_hbm.at[pl.ds(s * NPT, NPT), pl.ds(0, D_EDGE)])

    return sc_scatter


_sc_scatter = _sc_scatter_build()


def _mlp_a_body(x_ref, bat_ref, u_ref, w0_ref, b0_ref, t_ref):
    f32 = jnp.float32
    uw = jnp.dot(u_ref[...], w0_ref[D_FEAT + D_EDGE:, :],
                 preferred_element_type=f32)
    onehot = (bat_ref[...] == lax.broadcasted_iota(jnp.int32, (BN, N_GRAPHS), 1)
              ).astype(f32)
    t = jnp.dot(x_ref[...], w0_ref[:D_FEAT, :], preferred_element_type=f32)
    t_ref[...] = t + jnp.dot(onehot, uw, preferred_element_type=f32) + b0_ref[...]


def _mlp_b_body(t_ref, s0_ref, s1_ref, w0_ref, w1_ref, b1_ref, w2_ref, b2_ref,
                g_ref, bb_ref, o_ref):
    f32 = jnp.float32
    sE = s0_ref[..., :D_EDGE] + s1_ref[..., :D_EDGE]
    h = t_ref[...] + jnp.dot(sE, w0_ref[D_FEAT:D_FEAT + D_EDGE, :],
                             preferred_element_type=f32)
    h = jnp.maximum(h, 0.0)
    h = jnp.maximum(jnp.dot(h, w1_ref[...], preferred_element_type=f32) + b1_ref[...], 0.0)
    h = jnp.maximum(jnp.dot(h, w2_ref[...], preferred_element_type=f32) + b2_ref[...], 0.0)
    mu = jnp.mean(h, axis=1, keepdims=True)
    var = jnp.mean((h - mu) * (h - mu), axis=1, keepdims=True)
    o_ref[...] = (h - mu) * lax.rsqrt(var + 1e-5) * g_ref[...] + bb_ref[...]


def _mlp_call(x, s0, s1, batch2d, u, W0, b0, W1, b1, W2, b2, g, bb):
    grid = (N_NODES // BN,)
    full = lambda shape: pl.BlockSpec(shape, lambda i: (0, 0))
    t = pl.pallas_call(
        _mlp_a_body,
        grid=grid,
        in_specs=[
            pl.BlockSpec((BN, D_FEAT), lambda i: (i, 0)),
            pl.BlockSpec((BN, 1), lambda i: (i, 0)),
            full((N_GRAPHS, D_GLOBAL)),
            full((D_FEAT + D_EDGE + D_GLOBAL, H1)),
            full((1, H1)),
        ],
        out_specs=pl.BlockSpec((BN, H1), lambda i: (i, 0)),
        out_shape=jax.ShapeDtypeStruct((N_NODES, H1), jnp.float32),
    )(x, batch2d, u, W0, b0)
    return pl.pallas_call(
        _mlp_b_body,
        grid=grid,
        in_specs=[
            pl.BlockSpec((BN, H1), lambda i: (i, 0)),
            pl.BlockSpec((BN, 128), lambda i: (i, 0)),
            pl.BlockSpec((BN, 128), lambda i: (i, 0)),
            full((D_FEAT + D_EDGE + D_GLOBAL, H1)),
            full((H1, H2)),
            full((1, H2)),
            full((H2, N_OUT)),
            full((1, N_OUT)),
            full((1, N_OUT)),
            full((1, N_OUT)),
        ],
        out_specs=pl.BlockSpec((BN, N_OUT), lambda i: (i, 0)),
        out_shape=jax.ShapeDtypeStruct((N_NODES, N_OUT), jnp.float32),
    )(t, s0, s1, W0, W1, b1, W2, b2, g, bb)


def kernel(x, e, u, edge_index, batch, W0, b0, W1, b1, W2, b2, ln_scale, ln_bias):
    ev = e.T.reshape(2, 8, N_ROWS, EPR).transpose(0, 2, 1, 3)
    ei = edge_index.reshape(2, N_ROWS, EPR).transpose(1, 0, 2)
    s0, s1 = _sc_scatter(ev, ei)
    return _mlp_call(
        x, s0, s1, batch[:, None], u, W0, b0[None, :],
        W1, b1[None, :], W2, b2[None, :], ln_scale[None, :], ln_bias[None, :])


# speedup vs baseline: 1.1230x; 1.1230x over previous
import functools

import jax
import jax.numpy as jnp
from jax import lax
from jax.experimental import pallas as pl
from jax.experimental.pallas import tpu as pltpu
from jax.experimental.pallas import tpu_sc as plsc

N_NODES = 10000
N_EDGES = 320000
D_FEAT = 128
D_EDGE = 16
N_GRAPHS = 16
D_GLOBAL = 32
H1 = 256
H2 = 256
N_OUT = 128

BN = 2000

NC = 2
NS = 16
NW = NC * NS
EPR = 128
N_ROWS = N_EDGES // EPR
K_FULL = N_ROWS // NW
N_EXTRA = N_ROWS - K_FULL * NW
NPT = 1000


def _sc_scatter_build():
    mesh = plsc.VectorSubcoreMesh(core_axis_name="c", subcore_axis_name="s")
    NBUF = 13
    AHEAD = 10

    @functools.partial(
        pl.kernel, mesh=mesh,
        compiler_params=pltpu.CompilerParams(
            use_tc_tiling_on_sc=False, needs_layout_passes=False),
        out_type=(jax.ShapeDtypeStruct((N_NODES, 128), jnp.float32),
                  jax.ShapeDtypeStruct((N_NODES, 128), jnp.float32)),
        scratch_types=(
            [pltpu.VMEM((EPR,), jnp.int32) for _ in range(NBUF)]
            + [pltpu.VMEM((2, 8, EPR), jnp.float32) for _ in range(NBUF)]
            + [pltpu.VMEM((EPR, D_EDGE), jnp.float32) for _ in range(NBUF)]
            + [pltpu.VMEM((NPT, D_EDGE), jnp.float32),
               pltpu.VMEM_SHARED((N_NODES, D_EDGE), jnp.float32)]
            + [pltpu.SemaphoreType.DMA for _ in range(2 * NBUF)]
        ))
    def sc_scatter(ev_hbm, ei_hbm, out0_hbm, out1_hbm, *bufs):
        idx = bufs[0:NBUF]
        tb = bufs[NBUF:2 * NBUF]
        sb = bufs[2 * NBUF:3 * NBUF]
        stage = bufs[3 * NBUF]
        acc = bufs[3 * NBUF + 1]
        sl = bufs[3 * NBUF + 2:3 * NBUF + 2 + NBUF]
        ss = bufs[3 * NBUF + 2 + NBUF:3 * NBUF + 2 + 2 * NBUF]

        c = lax.axis_index("c")
        s = lax.axis_index("s")
        w = s * NC + c
        iota16 = lax.broadcasted_iota(jnp.int32, (D_EDGE,), 0)

        def zrow(i, carry):
            stage[i, :] = jnp.zeros((D_EDGE,), jnp.float32)
            return carry
        lax.fori_loop(0, NPT, zrow, 0)

        @pl.when(s < N_NODES // NPT)
        def _():
            pltpu.sync_copy(stage, acc.at[pl.ds(s * NPT, NPT)])
        plsc.subcore_barrier()

        def start_load(k, b):
            r = w + k * NW
            pltpu.async_copy(ei_hbm.at[r, 1], idx[b], sl[b])
            pltpu.async_copy(ev_hbm.at[:, r], tb[b], sl[b])

        def wait_load(b):
            pltpu.make_async_copy(ei_hbm.at[0, 1], idx[b], sl[b]).wait()
            pltpu.make_async_copy(ev_hbm.at[:, 0], tb[b], sl[b]).wait()

        rot = [((iota16 + d) & 15) for d in range(D_EDGE)]
        hi3 = iota16 >> 3
        lo3 = iota16 & 7
        zv = jnp.zeros((D_EDGE,), jnp.int32)
        ldb = [hi3 * (8 * EPR) + lo3 * EPR + rot[d] for d in range(D_EDGE)]
        stb = [rot[d] * D_EDGE + iota16 for d in range(D_EDGE)]

        def transpose(b):
            def tr(m, carry):
                vs = [plsc.load_gather(tb[b], [zv, zv, ldb[d] + m * D_EDGE])
                      for d in range(D_EDGE)]
                for d in range(D_EDGE):
                    plsc.store_scatter(
                        sb[b], [zv, stb[d] + m * (D_EDGE * D_EDGE)], vs[d])
                return carry
            lax.fori_loop(0, EPR // D_EDGE, tr, 0)

        def start_scat(b):
            pltpu.async_copy(sb[b], acc.at[idx[b]], ss[b], add=True)

        def wait_scat(b):
            pltpu.make_async_copy(sb[b], acc.at[idx[b]], ss[b]).wait()

        for b in range(AHEAD):
            start_load(b, b)

        def body(g, carry):
            for i in range(NBUF):
                b = i
                k = g * NBUF + i

                wait_load(b)
                transpose(b)
                start_scat(b)

                bb = (i + AHEAD) % NBUF

                @pl.when(k + AHEAD < K_FULL)
                def _():
                    @pl.when(k + AHEAD >= NBUF)
                    def _():
                        wait_scat(bb)
                    start_load(k + AHEAD, bb)
            return carry

        lax.fori_loop(0, K_FULL // NBUF, body, 0)

        @pl.when(w < N_EXTRA)
        def _():
            wait_scat(0)
            start_load(K_FULL, 0)
            wait_load(0)
            transpose(0)
            start_scat(0)

        for b in range(NBUF):
            wait_scat(b)

        plsc.subcore_barrier()

        @pl.when(s < N_NODES // NPT)
        def _():
            src_slice = acc.at[pl.ds(s * NPT, NPT)]

            @pl.when(c == 0)
            def _():
                pltpu.sync_copy(src_slice,
                                out0_hbm.at[pl.ds(s * NPT, NPT), pl.ds(0, D_EDGE)])

            @pl.when(c == 1)
            def _():
                pltpu.sync_copy(src_slice,
                                out1_hbm.at[pl.ds(s * NPT, NPT), pl.ds(0, D_EDGE)])

    return sc_scatter


_sc_scatter = _sc_scatter_build()


def _mlp_a_body(x_ref, bat_ref, u_ref, w0_ref, b0_ref, t_ref):
    f32 = jnp.float32
    uw = jnp.dot(u_ref[...], w0_ref[D_FEAT + D_EDGE:, :],
                 preferred_element_type=f32)
    onehot = (bat_ref[...] == lax.broadcasted_iota(jnp.int32, (BN, N_GRAPHS), 1)
              ).astype(f32)
    t = jnp.dot(x_ref[...], w0_ref[:D_FEAT, :], preferred_element_type=f32)
    t_ref[...] = t + jnp.dot(onehot, uw, preferred_element_type=f32) + b0_ref[...]


def _mlp_b_body(t_ref, s0_ref, s1_ref, w0_ref, w1_ref, b1_ref, w2_ref, b2_ref,
                g_ref, bb_ref, o_ref):
    f32 = jnp.float32
    sE = s0_ref[..., :D_EDGE] + s1_ref[..., :D_EDGE]
    h = t_ref[...] + jnp.dot(sE, w0_ref[D_FEAT:D_FEAT + D_EDGE, :],
                             preferred_element_type=f32)
    h = jnp.maximum(h, 0.0)
    h = jnp.maximum(jnp.dot(h, w1_ref[...], preferred_element_type=f32) + b1_ref[...], 0.0)
    h = jnp.maximum(jnp.dot(h, w2_ref[...], preferred_element_type=f32) + b2_ref[...], 0.0)
    mu = jnp.mean(h, axis=1, keepdims=True)
    var = jnp.mean((h - mu) * (h - mu), axis=1, keepdims=True)
    o_ref[...] = (h - mu) * lax.rsqrt(var + 1e-5) * g_ref[...] + bb_ref[...]


def _mlp_call(x, s0, s1, batch2d, u, W0, b0, W1, b1, W2, b2, g, bb):
    grid = (N_NODES // BN,)
    full = lambda shape: pl.BlockSpec(shape, lambda i: (0, 0))
    t = pl.pallas_call(
        _mlp_a_body,
        grid=grid,
        in_specs=[
            pl.BlockSpec((BN, D_FEAT), lambda i: (i, 0)),
            pl.BlockSpec((BN, 1), lambda i: (i, 0)),
            full((N_GRAPHS, D_GLOBAL)),
            full((D_FEAT + D_EDGE + D_GLOBAL, H1)),
            full((1, H1)),
        ],
        out_specs=pl.BlockSpec((BN, H1), lambda i: (i, 0)),
        out_shape=jax.ShapeDtypeStruct((N_NODES, H1), jnp.float32),
    )(x, batch2d, u, W0, b0)
    return pl.pallas_call(
        _mlp_b_body,
        grid=grid,
        in_specs=[
            pl.BlockSpec((BN, H1), lambda i: (i, 0)),
            pl.BlockSpec((BN, 128), lambda i: (i, 0)),
            pl.BlockSpec((BN, 128), lambda i: (i, 0)),
            full((D_FEAT + D_EDGE + D_GLOBAL, H1)),
            full((H1, H2)),
            full((1, H2)),
            full((H2, N_OUT)),
            full((1, N_OUT)),
            full((1, N_OUT)),
            full((1, N_OUT)),
        ],
        out_specs=pl.BlockSpec((BN, N_OUT), lambda i: (i, 0)),
        out_shape=jax.ShapeDtypeStruct((N_NODES, N_OUT), jnp.float32),
    )(t, s0, s1, W0, W1, b1, W2, b2, g, bb)


def kernel(x, e, u, edge_index, batch, W0, b0, W1, b1, W2, b2, ln_scale, ln_bias):
    ev = e.T.reshape(2, 8, N_ROWS, EPR).transpose(0, 2, 1, 3)
    ei = edge_index.reshape(2, N_ROWS, EPR).transpose(1, 0, 2)
    s0, s1 = _sc_scatter(ev, ei)
    return _mlp_call(
        x, s0, s1, batch[:, None], u, W0, b0[None, :],
        W1, b1[None, :], W2, b2[None, :], ln_scale[None, :], ln_bias[None, :])
